# baseline (speedup 1.0000x reference)
import functools

import jax
import jax.numpy as jnp
from jax import lax
from jax.experimental import pallas as pl
from jax.experimental.pallas import tpu as pltpu
from jax.experimental.pallas import tpu_sc as plsc

DIM = 256
NUM_HEADS = 4
HEAD_DIM = DIM // NUM_HEADS
TOP_K = 16
TEMPERATURE = 0.1
B = 1024
P = 10000
PPAD = 10240

NC = 2
NS = 16
LANES = 16
NW = NC * NS
ROWS_PER_W = B // NW

PTILE = 1024
NPT = PPAD // PTILE

NEG = -3e38
SCALE = 1.0 / (NUM_HEADS * TEMPERATURE)


def _head_selector():
  d = lax.broadcasted_iota(jnp.int32, (DIM, NUM_HEADS), 0)
  h = lax.broadcasted_iota(jnp.int32, (DIM, NUM_HEADS), 1)
  return (d // HEAD_DIM == h).astype(jnp.float32)


def _head_normalize(x, sel, scale):
  ss = lax.dot_general(x * x, sel, (((1,), (0,)), ((), ())),
                       preferred_element_type=jnp.float32,
                       precision=lax.Precision.HIGHEST)
  inv = scale / jnp.maximum(jnp.sqrt(ss), 1e-12)
  inv_full = lax.dot_general(inv, sel, (((1,), (1,)), ((), ())),
                             preferred_element_type=jnp.float32,
                       precision=lax.Precision.HIGHEST)
  return x * inv_full


def _tc_body(h_ref, protos_ref, wqr_ref, wkr_ref, wout_ref,
             scores_ref, pout_ref, qn_scr):
  pid = pl.program_id(0)
  sel = _head_selector()

  @pl.when(pid == 0)
  def _():
    qr = lax.dot_general(h_ref[...], wqr_ref[...], (((1,), (1,)), ((), ())),
                         preferred_element_type=jnp.float32)
    qn_scr[...] = _head_normalize(qr, sel, 1.0)

  protos = protos_ref[...]
  kr = lax.dot_general(protos, wkr_ref[...], (((1,), (1,)), ((), ())),
                       preferred_element_type=jnp.float32)
  kn = _head_normalize(kr, sel, 1.0)
  s = lax.dot_general(qn_scr[...], kn, (((1,), (1,)), ((), ())),
                      preferred_element_type=jnp.float32)
  col = pid * PTILE + lax.broadcasted_iota(jnp.int32, (B, PTILE), 1)
  scores_ref[...] = jnp.where(col < P, s * SCALE, NEG)
  pout_ref[...] = lax.dot_general(protos, wout_ref[...], (((1,), (1,)), ((), ())),
                                  preferred_element_type=jnp.float32)


def _tc_scores(h, protos_pad, w_qr, w_kr, w_out):
  return pl.pallas_call(
      _tc_body,
      grid=(NPT,),
      in_specs=[
          pl.BlockSpec((B, DIM), lambda i: (0, 0)),
          pl.BlockSpec((PTILE, DIM), lambda i: (i, 0)),
          pl.BlockSpec((DIM, DIM), lambda i: (0, 0)),
          pl.BlockSpec((DIM, DIM), lambda i: (0, 0)),
          pl.BlockSpec((DIM, DIM), lambda i: (0, 0)),
      ],
      out_specs=[
          pl.BlockSpec((B, PTILE), lambda i: (0, i)),
          pl.BlockSpec((PTILE, DIM), lambda i: (i, 0)),
      ],
      out_shape=[
          jax.ShapeDtypeStruct((B, PPAD), jnp.float32),
          jax.ShapeDtypeStruct((PPAD, DIM), jnp.float32),
      ],
      scratch_shapes=[pltpu.VMEM((B, DIM), jnp.float32)],
      compiler_params=pltpu.CompilerParams(
          dimension_semantics=("arbitrary",)),
  )(h, protos_pad, w_qr, w_kr, w_out)


def _sc_topk_gather(scores, pout):
  mesh = plsc.VectorSubcoreMesh(
      core_axis_name="c", subcore_axis_name="s",
      num_cores=NC, num_subcores=NS)

  @functools.partial(
      pl.kernel,
      mesh=mesh,
      out_type=jax.ShapeDtypeStruct((B, DIM), jnp.float32),
      compiler_params=pltpu.CompilerParams(needs_layout_passes=False),
      scratch_types=[
          pltpu.VMEM((2, PPAD), jnp.float32),
          pltpu.VMEM((TOP_K,), jnp.int32),
          pltpu.VMEM((TOP_K, DIM), jnp.float32),
          pltpu.VMEM((ROWS_PER_W, DIM), jnp.float32),
          pltpu.SemaphoreType.DMA,
          pltpu.SemaphoreType.DMA,
          pltpu.SemaphoreType.DMA,
      ],
  )
  def sc_kernel(scores_hbm, pout_hbm, out_hbm,
                row_buf, idx_v, rows_v, out_buf, sem0, sem1, sem_g):
    wid = lax.axis_index("s") * NC + lax.axis_index("c")
    base = wid * ROWS_PER_W
    sems = (sem0, sem1)

    pltpu.async_copy(scores_hbm.at[base], row_buf.at[0], sem0)
    pltpu.async_copy(scores_hbm.at[base + 1], row_buf.at[1], sem1)

    def topk_row(slot):
      def chunk(c, carry):
        a_val, a_idx = carry
        v = row_buf[slot, pl.ds(c * LANES, LANES)]
        hit = jnp.any(v > a_val[0])

        def merge(carry):
          a_val, a_idx = carry
          idx = c * LANES + lax.iota(jnp.int32, LANES)
          vd, idxd = plsc.sort_key_val(v, idx, descending=True)
          take = vd > a_val
          nv = jnp.where(take, vd, a_val)
          ni = jnp.where(take, idxd, a_idx)
          sv, si = plsc.sort_key_val(nv, ni, descending=False)
          return (sv, si)

        return lax.cond(hit, merge, lambda carry: carry, (a_val, a_idx))

      a0 = jnp.full((LANES,), NEG, jnp.float32)
      i0 = jnp.zeros((LANES,), jnp.int32)
      return lax.fori_loop(0, PPAD // LANES, chunk, (a0, i0))

    def do_row(j, slot):
      pltpu.make_async_copy(
          scores_hbm.at[base], row_buf.at[slot], sems[slot]).wait()

      a_val, a_idx = topk_row(slot)

      @pl.when(j + 2 < ROWS_PER_W)
      def _():
        pltpu.async_copy(
            scores_hbm.at[base + j + 2], row_buf.at[slot], sems[slot])

      e = jnp.exp(a_val - jnp.max(a_val))
      w = e / jnp.sum(e)

      idx_v[...] = a_idx
      pltpu.async_copy(pout_hbm.at[idx_v], rows_v, sem_g).wait()

      acc = [jnp.zeros((LANES,), jnp.float32) for _ in range(DIM // LANES)]
      for k in range(TOP_K):
        wk = w[k]
        for d in range(DIM // LANES):
          acc[d] = acc[d] + wk * rows_v[k, pl.ds(d * LANES, LANES)]
      for d in range(DIM // LANES):
        out_buf[j, pl.ds(d * LANES, LANES)] = acc[d]

    def pair(g, _):
      do_row(g, 0)
      do_row(g + 1, 1)
      return 0

    lax.fori_loop(0, ROWS_PER_W // 2, lambda t, c: pair(2 * t, c), 0,
                  unroll=False)

    pltpu.sync_copy(out_buf, out_hbm.at[pl.ds(base, ROWS_PER_W)])

  return sc_kernel(scores, pout)


def kernel(h, prototypes, W_QR, W_KR, W_out):
  protos_pad = jnp.pad(prototypes, ((0, PPAD - P), (0, 0)))
  scores, pout = _tc_scores(h, protos_pad, W_QR, W_KR, W_out)
  return _sc_topk_gather(scores, pout)


# speedup vs baseline: 3.0077x; 3.0077x over previous
import functools

import jax
import jax.numpy as jnp
from jax import lax
from jax.experimental import pallas as pl
from jax.experimental.pallas import tpu as pltpu
from jax.experimental.pallas import tpu_sc as plsc

DIM = 256
NUM_HEADS = 4
HEAD_DIM = DIM // NUM_HEADS
TOP_K = 16
TEMPERATURE = 0.1
B = 1024
P = 10000
PPAD = 10240

NC = 2
NS = 16
LANES = 16
NW = NC * NS
ROWS_PER_W = B // NW

PTILE = 1024
NPT = PPAD // PTILE
GROUP = 128
NG = PPAD // GROUP

NEG = -3e38
SCALE = 1.0 / (NUM_HEADS * TEMPERATURE)


def _head_selector():
  d = lax.broadcasted_iota(jnp.int32, (DIM, NUM_HEADS), 0)
  h = lax.broadcasted_iota(jnp.int32, (DIM, NUM_HEADS), 1)
  return (d // HEAD_DIM == h).astype(jnp.float32)


def _head_normalize(x, sel, scale):
  ss = lax.dot_general(x * x, sel, (((1,), (0,)), ((), ())),
                       preferred_element_type=jnp.float32,
                       precision=lax.Precision.HIGHEST)
  inv = scale / jnp.maximum(jnp.sqrt(ss), 1e-12)
  inv_full = lax.dot_general(inv, sel, (((1,), (1,)), ((), ())),
                             preferred_element_type=jnp.float32,
                       precision=lax.Precision.HIGHEST)
  return x * inv_full


def _tc_body(h_ref, protos_ref, wqr_ref, wkr_ref, wout_ref,
             scores_ref, pout_ref, gmax_ref, qn_scr):
  pid = pl.program_id(0)
  sel = _head_selector()

  @pl.when(pid == 0)
  def _():
    qr = lax.dot_general(h_ref[...], wqr_ref[...], (((1,), (1,)), ((), ())),
                         preferred_element_type=jnp.float32)
    qn_scr[...] = _head_normalize(qr, sel, 1.0)

  protos = protos_ref[...]
  kr = lax.dot_general(protos, wkr_ref[...], (((1,), (1,)), ((), ())),
                       preferred_element_type=jnp.float32)
  kn = _head_normalize(kr, sel, 1.0)
  s = lax.dot_general(qn_scr[...], kn, (((1,), (1,)), ((), ())),
                      preferred_element_type=jnp.float32)
  col = pid * PTILE + lax.broadcasted_iota(jnp.int32, (B, PTILE), 1)
  s = jnp.where(col < P, s * SCALE, NEG)
  scores_ref[...] = s
  gmax_ref[0] = jnp.concatenate(
      [jnp.max(s[:, g * GROUP:(g + 1) * GROUP], axis=1, keepdims=True)
       for g in range(PTILE // GROUP)], axis=1)
  pout_ref[...] = lax.dot_general(protos, wout_ref[...], (((1,), (1,)), ((), ())),
                                  preferred_element_type=jnp.float32)


def _tc_scores(h, protos_pad, w_qr, w_kr, w_out):
  return pl.pallas_call(
      _tc_body,
      grid=(NPT,),
      in_specs=[
          pl.BlockSpec((B, DIM), lambda i: (0, 0)),
          pl.BlockSpec((PTILE, DIM), lambda i: (i, 0)),
          pl.BlockSpec((DIM, DIM), lambda i: (0, 0)),
          pl.BlockSpec((DIM, DIM), lambda i: (0, 0)),
          pl.BlockSpec((DIM, DIM), lambda i: (0, 0)),
      ],
      out_specs=[
          pl.BlockSpec((B, PTILE), lambda i: (0, i)),
          pl.BlockSpec((PTILE, DIM), lambda i: (i, 0)),
          pl.BlockSpec((1, B, PTILE // GROUP), lambda i: (i, 0, 0)),
      ],
      out_shape=[
          jax.ShapeDtypeStruct((B, PPAD), jnp.float32),
          jax.ShapeDtypeStruct((PPAD, DIM), jnp.float32),
          jax.ShapeDtypeStruct((NPT, B, PTILE // GROUP), jnp.float32),
      ],
      scratch_shapes=[pltpu.VMEM((B, DIM), jnp.float32)],
      compiler_params=pltpu.CompilerParams(
          dimension_semantics=("arbitrary",)),
  )(h, protos_pad, w_qr, w_kr, w_out)


def _sc_topk_gather(scores, pout, gmax):
  mesh = plsc.VectorSubcoreMesh(
      core_axis_name="c", subcore_axis_name="s",
      num_cores=NC, num_subcores=NS)

  @functools.partial(
      pl.kernel,
      mesh=mesh,
      out_type=jax.ShapeDtypeStruct((B, DIM), jnp.float32),
      compiler_params=pltpu.CompilerParams(needs_layout_passes=False),
      scratch_types=[
          pltpu.VMEM((2, PPAD), jnp.float32),
          pltpu.VMEM((2, NG), jnp.float32),
          pltpu.VMEM((TOP_K,), jnp.int32),
          pltpu.VMEM((TOP_K, DIM), jnp.float32),
          pltpu.VMEM((ROWS_PER_W, DIM), jnp.float32),
          pltpu.SemaphoreType.DMA,
          pltpu.SemaphoreType.DMA,
          pltpu.SemaphoreType.DMA,
      ],
  )
  def sc_kernel(scores_hbm, pout_hbm, gmax_hbm, out_hbm,
                row_buf, gm_buf, idx_v, rows_v, out_buf, sem0, sem1, sem_g):
    wid = lax.axis_index("s") * NC + lax.axis_index("c")
    base = wid * ROWS_PER_W
    sems = (sem0, sem1)

    def fetch(r, slot):
      pltpu.async_copy(scores_hbm.at[r], row_buf.at[slot], sems[slot])
      pltpu.async_copy(gmax_hbm.at[r], gm_buf.at[slot], sems[slot])

    fetch(base, 0)
    fetch(base + 1, 1)

    def topk_row(slot):
      t = lax.sort(gm_buf[slot, pl.ds(0, LANES)])
      for c in range(1, NG // LANES):
        g = gm_buf[slot, pl.ds(c * LANES, LANES)]
        gd, _ = plsc.sort_key_val(g, g, descending=True)
        t = lax.sort(jnp.maximum(t, gd))
      thresh0 = t[0]

      a_val = jnp.full((LANES,), NEG, jnp.float32)
      a_idx = jnp.zeros((LANES,), jnp.int32)
      lanes = lax.iota(jnp.int32, LANES)

      for c in range(NG // LANES):
        gvec = gm_buf[slot, pl.ds(c * LANES, LANES)]
        mask0 = gvec >= thresh0

        def body_fn(carry, c=c):
          mask, a_val, a_idx = carry
          gl = plsc.all_reduce_ffs(mask)[0]
          col0 = (c * LANES + gl) * GROUP
          for cc in range(GROUP // LANES):
            v = row_buf[slot, pl.ds(col0 + cc * LANES, LANES)]
            hit = jnp.any(v > a_val[0])

            def merge(carry, v=v, cc=cc):
              a_val, a_idx = carry
              idx = (col0 + cc * LANES) + lanes
              vd, idxd = plsc.sort_key_val(v, idx, descending=True)
              take = vd > a_val
              nv = jnp.where(take, vd, a_val)
              ni = jnp.where(take, idxd, a_idx)
              sv, si = plsc.sort_key_val(nv, ni, descending=False)
              return (sv, si)

            a_val, a_idx = lax.cond(hit, merge, lambda cr: cr, (a_val, a_idx))
          return (mask & (lanes != gl), a_val, a_idx)

        _, a_val, a_idx = lax.while_loop(
            lambda carry: jnp.any(carry[0]), body_fn, (mask0, a_val, a_idx))
      return a_val, a_idx

    def do_row(j, slot):
      pltpu.make_async_copy(
          scores_hbm.at[base], row_buf.at[slot], sems[slot]).wait()
      pltpu.make_async_copy(
          gmax_hbm.at[base], gm_buf.at[slot], sems[slot]).wait()

      a_val, a_idx = topk_row(slot)

      @pl.when(j + 2 < ROWS_PER_W)
      def _():
        fetch(base + j + 2, slot)

      e = jnp.exp(a_val - jnp.max(a_val))
      w = e / jnp.sum(e)

      idx_v[...] = a_idx
      pltpu.async_copy(pout_hbm.at[idx_v], rows_v, sem_g).wait()

      acc = [jnp.zeros((LANES,), jnp.float32) for _ in range(DIM // LANES)]
      for k in range(TOP_K):
        wk = w[k]
        for d in range(DIM // LANES):
          acc[d] = acc[d] + wk * rows_v[k, pl.ds(d * LANES, LANES)]
      for d in range(DIM // LANES):
        out_buf[j, pl.ds(d * LANES, LANES)] = acc[d]

    def pair(g, _):
      do_row(g, 0)
      do_row(g + 1, 1)
      return 0

    lax.fori_loop(0, ROWS_PER_W // 2, lambda t, c: pair(2 * t, c), 0,
                  unroll=False)

    pltpu.sync_copy(out_buf, out_hbm.at[pl.ds(base, ROWS_PER_W)])

  return sc_kernel(scores, pout, gmax)


def kernel(h, prototypes, W_QR, W_KR, W_out):
  protos_pad = jnp.pad(prototypes, ((0, PPAD - P), (0, 0)))
  scores, pout, gmax3 = _tc_scores(h, protos_pad, W_QR, W_KR, W_out)
  gmax = gmax3.transpose(1, 0, 2).reshape(B, NG)
  return _sc_topk_gather(scores, pout, gmax)


# speedup vs baseline: 4.6439x; 1.5440x over previous
import functools

import jax
import jax.numpy as jnp
from jax import lax
from jax.experimental import pallas as pl
from jax.experimental.pallas import tpu as pltpu
from jax.experimental.pallas import tpu_sc as plsc

DIM = 256
NUM_HEADS = 4
HEAD_DIM = DIM // NUM_HEADS
TOP_K = 16
TEMPERATURE = 0.1
B = 1024
P = 10000
PPAD = 10240

NC = 2
NS = 16
LANES = 16
NW = NC * NS
ROWS_PER_W = B // NW

PTILE = 1024
NPT = PPAD // PTILE
GROUP = 128
NG = PPAD // GROUP

NEG = -3e38
SCALE = 1.0 / (NUM_HEADS * TEMPERATURE)


def _head_selector():
  d = lax.broadcasted_iota(jnp.int32, (DIM, NUM_HEADS), 0)
  h = lax.broadcasted_iota(jnp.int32, (DIM, NUM_HEADS), 1)
  return (d // HEAD_DIM == h).astype(jnp.float32)


def _head_normalize(x, sel, scale):
  ss = lax.dot_general(x * x, sel, (((1,), (0,)), ((), ())),
                       preferred_element_type=jnp.float32,
                       precision=lax.Precision.HIGHEST)
  inv = scale / jnp.maximum(jnp.sqrt(ss), 1e-12)
  inv_full = lax.dot_general(inv, sel, (((1,), (1,)), ((), ())),
                             preferred_element_type=jnp.float32,
                       precision=lax.Precision.HIGHEST)
  return x * inv_full


def _tc_body(h_ref, protos_ref, wqr_ref, wkr_ref, wout_ref,
             scores_ref, pout_ref, gmax_ref, qn_scr):
  pid = pl.program_id(0)
  sel = _head_selector()

  @pl.when(pid == 0)
  def _():
    qr = lax.dot_general(h_ref[...], wqr_ref[...], (((1,), (1,)), ((), ())),
                         preferred_element_type=jnp.float32)
    qn_scr[...] = _head_normalize(qr, sel, 1.0)

  protos = protos_ref[...]
  kr = lax.dot_general(protos, wkr_ref[...], (((1,), (1,)), ((), ())),
                       preferred_element_type=jnp.float32)
  kn = _head_normalize(kr, sel, 1.0)
  s = lax.dot_general(qn_scr[...], kn, (((1,), (1,)), ((), ())),
                      preferred_element_type=jnp.float32)
  col = pid * PTILE + lax.broadcasted_iota(jnp.int32, (B, PTILE), 1)
  s = jnp.where(col < P, s * SCALE, NEG)
  scores_ref[...] = s
  gmax_ref[0] = jnp.concatenate(
      [jnp.max(s[:, g * GROUP:(g + 1) * GROUP], axis=1, keepdims=True)
       for g in range(PTILE // GROUP)], axis=1)
  pout_ref[...] = lax.dot_general(protos, wout_ref[...], (((1,), (1,)), ((), ())),
                                  preferred_element_type=jnp.float32)


def _tc_scores(h, protos_pad, w_qr, w_kr, w_out):
  return pl.pallas_call(
      _tc_body,
      grid=(NPT,),
      in_specs=[
          pl.BlockSpec((B, DIM), lambda i: (0, 0)),
          pl.BlockSpec((PTILE, DIM), lambda i: (i, 0)),
          pl.BlockSpec((DIM, DIM), lambda i: (0, 0)),
          pl.BlockSpec((DIM, DIM), lambda i: (0, 0)),
          pl.BlockSpec((DIM, DIM), lambda i: (0, 0)),
      ],
      out_specs=[
          pl.BlockSpec((B, PTILE), lambda i: (0, i)),
          pl.BlockSpec((PTILE, DIM), lambda i: (i, 0)),
          pl.BlockSpec((1, B, PTILE // GROUP), lambda i: (i, 0, 0)),
      ],
      out_shape=[
          jax.ShapeDtypeStruct((B, PPAD), jnp.float32),
          jax.ShapeDtypeStruct((PPAD, DIM), jnp.float32),
          jax.ShapeDtypeStruct((NPT, B, PTILE // GROUP), jnp.float32),
      ],
      scratch_shapes=[pltpu.VMEM((B, DIM), jnp.float32)],
      compiler_params=pltpu.CompilerParams(
          dimension_semantics=("arbitrary",)),
  )(h, protos_pad, w_qr, w_kr, w_out)


def _sc_topk_gather(scores, pout, gmax):
  mesh = plsc.VectorSubcoreMesh(
      core_axis_name="c", subcore_axis_name="s",
      num_cores=NC, num_subcores=NS)

  @functools.partial(
      pl.kernel,
      mesh=mesh,
      out_type=jax.ShapeDtypeStruct((B, DIM), jnp.float32),
      compiler_params=pltpu.CompilerParams(needs_layout_passes=False),
      scratch_types=[
          pltpu.VMEM((2, PPAD), jnp.float32),
          pltpu.VMEM((2, NG), jnp.float32),
          pltpu.VMEM((PPAD + LANES,), jnp.float32),
          pltpu.VMEM((PPAD + LANES,), jnp.int32),
          pltpu.VMEM((TOP_K,), jnp.int32),
          pltpu.VMEM((TOP_K, DIM), jnp.float32),
          pltpu.VMEM((ROWS_PER_W, DIM), jnp.float32),
          pltpu.SemaphoreType.DMA,
          pltpu.SemaphoreType.DMA,
          pltpu.SemaphoreType.DMA,
      ],
  )
  def sc_kernel(scores_hbm, pout_hbm, gmax_hbm, out_hbm,
                row_buf, gm_buf, cand_val, cand_idx, idx_v, rows_v, out_buf,
                sem0, sem1, sem_g):
    wid = lax.axis_index("s") * NC + lax.axis_index("c")
    base = wid * ROWS_PER_W
    sems = (sem0, sem1)

    def fetch(r, slot):
      pltpu.async_copy(scores_hbm.at[r], row_buf.at[slot], sems[slot])
      pltpu.async_copy(gmax_hbm.at[r], gm_buf.at[slot], sems[slot])

    fetch(base, 0)
    fetch(base + 1, 1)

    def topk_row(slot):
      t = lax.sort(gm_buf[slot, pl.ds(0, LANES)])
      for c in range(1, NG // LANES):
        g = gm_buf[slot, pl.ds(c * LANES, LANES)]
        gd, _ = plsc.sort_key_val(g, g, descending=True)
        t = lax.sort(jnp.maximum(t, gd))
      thresh0 = t[0]

      lanes = lax.iota(jnp.int32, LANES)

      cnt = jnp.int32(0)
      for c in range(NG // LANES):
        gvec = gm_buf[slot, pl.ds(c * LANES, LANES)]
        mask0 = gvec >= thresh0

        def body_fn(carry, c=c):
          mask, cnt = carry
          gl = plsc.all_reduce_ffs(mask)[0]
          col0 = (c * LANES + gl) * GROUP
          for cc in range(GROUP // LANES):
            colc = col0 + cc * LANES
            v = row_buf[slot, pl.ds(colc, LANES)]
            m = v >= thresh0
            plsc.store_compressed(cand_val.at[pl.ds(cnt, LANES)], v, mask=m)
            plsc.store_compressed(cand_idx.at[pl.ds(cnt, LANES)],
                                  colc + lanes, mask=m)
            cnt = cnt + plsc.all_reduce_population_count(m)[0]
          return (mask & (lanes != gl), cnt)

        _, cnt = lax.while_loop(
            lambda carry: jnp.any(carry[0]), body_fn, (mask0, cnt))

      cand_val[pl.ds(cnt, LANES)] = jnp.full((LANES,), NEG, jnp.float32)

      def mbody(mi, carry):
        a_val, a_idx = carry
        v = cand_val[pl.ds(mi * LANES, LANES)]
        i = cand_idx[pl.ds(mi * LANES, LANES)]
        vd, idxd = plsc.sort_key_val(v, i, descending=True)
        take = vd > a_val
        nv = jnp.where(take, vd, a_val)
        ni = jnp.where(take, idxd, a_idx)
        sv, si = plsc.sort_key_val(nv, ni, descending=False)
        return (sv, si)

      a0 = jnp.full((LANES,), NEG, jnp.float32)
      i0 = jnp.zeros((LANES,), jnp.int32)
      nch = (cnt + LANES - 1) // LANES
      return lax.fori_loop(0, nch, mbody, (a0, i0))

    def do_row(j, slot):
      pltpu.make_async_copy(
          scores_hbm.at[base], row_buf.at[slot], sems[slot]).wait()
      pltpu.make_async_copy(
          gmax_hbm.at[base], gm_buf.at[slot], sems[slot]).wait()

      a_val, a_idx = topk_row(slot)

      @pl.when(j + 2 < ROWS_PER_W)
      def _():
        fetch(base + j + 2, slot)

      e = jnp.exp(a_val - jnp.max(a_val))
      w = e / jnp.sum(e)

      idx_v[...] = a_idx
      pltpu.async_copy(pout_hbm.at[idx_v], rows_v, sem_g).wait()

      acc = [jnp.zeros((LANES,), jnp.float32) for _ in range(DIM // LANES)]
      for k in range(TOP_K):
        wk = w[k]
        for d in range(DIM // LANES):
          acc[d] = acc[d] + wk * rows_v[k, pl.ds(d * LANES, LANES)]
      for d in range(DIM // LANES):
        out_buf[j, pl.ds(d * LANES, LANES)] = acc[d]

    def pair(g, _):
      do_row(g, 0)
      do_row(g + 1, 1)
      return 0

    lax.fori_loop(0, ROWS_PER_W // 2, lambda t, c: pair(2 * t, c), 0,
                  unroll=False)

    pltpu.sync_copy(out_buf, out_hbm.at[pl.ds(base, ROWS_PER_W)])

  return sc_kernel(scores, pout, gmax)


def kernel(h, prototypes, W_QR, W_KR, W_out):
  protos_pad = jnp.pad(prototypes, ((0, PPAD - P), (0, 0)))
  scores, pout, gmax3 = _tc_scores(h, protos_pad, W_QR, W_KR, W_out)
  gmax = gmax3.transpose(1, 0, 2).reshape(B, NG)
  return _sc_topk_gather(scores, pout, gmax)


# speedup vs baseline: 5.3084x; 1.1431x over previous
import functools

import jax
import jax.numpy as jnp
from jax import lax
from jax.experimental import pallas as pl
from jax.experimental.pallas import tpu as pltpu
from jax.experimental.pallas import tpu_sc as plsc

DIM = 256
NUM_HEADS = 4
HEAD_DIM = DIM // NUM_HEADS
TOP_K = 16
TEMPERATURE = 0.1
B = 1024
P = 10000
PPAD = 10240

NC = 2
NS = 16
LANES = 16
NW = NC * NS
ROWS_PER_W = B // NW

PTILE = 2048
NPT = PPAD // PTILE
GROUP = 128
NG = PPAD // GROUP

NEG = -3e38
SCALE = 1.0 / (NUM_HEADS * TEMPERATURE)


def _head_selector():
  d = lax.broadcasted_iota(jnp.int32, (DIM, NUM_HEADS), 0)
  h = lax.broadcasted_iota(jnp.int32, (DIM, NUM_HEADS), 1)
  return (d // HEAD_DIM == h).astype(jnp.float32)


def _head_normalize(x, sel, scale):
  ss = lax.dot_general(x * x, sel, (((1,), (0,)), ((), ())),
                       preferred_element_type=jnp.float32,
                       precision=lax.Precision.HIGHEST)
  inv = scale / jnp.maximum(jnp.sqrt(ss), 1e-12)
  inv_full = lax.dot_general(inv, sel, (((1,), (1,)), ((), ())),
                             preferred_element_type=jnp.float32,
                       precision=lax.Precision.HIGHEST)
  return x * inv_full


def _tc_body(h_ref, protos_ref, wqr_ref, wkr_ref, wout_ref,
             scores_ref, pout_ref, gmax_ref, qn_scr):
  pid = pl.program_id(0)
  sel = _head_selector()

  @pl.when(pid == 0)
  def _():
    qr = lax.dot_general(h_ref[...], wqr_ref[...], (((1,), (1,)), ((), ())),
                         preferred_element_type=jnp.float32)
    qn_scr[...] = _head_normalize(qr, sel, 1.0)

  protos = protos_ref[...]
  kr = lax.dot_general(protos, wkr_ref[...], (((1,), (1,)), ((), ())),
                       preferred_element_type=jnp.float32)
  kn = _head_normalize(kr, sel, 1.0)
  s = lax.dot_general(qn_scr[...], kn, (((1,), (1,)), ((), ())),
                      preferred_element_type=jnp.float32)
  col = pid * PTILE + lax.broadcasted_iota(jnp.int32, (B, PTILE), 1)
  s = jnp.where(col < P, s * SCALE, NEG)
  scores_ref[...] = s
  gmax_ref[0] = jnp.concatenate(
      [jnp.max(s[:, g * GROUP:(g + 1) * GROUP], axis=1, keepdims=True)
       for g in range(PTILE // GROUP)], axis=1)
  pout_ref[...] = lax.dot_general(protos, wout_ref[...], (((1,), (1,)), ((), ())),
                                  preferred_element_type=jnp.float32)


def _tc_scores(h, protos_pad, w_qr, w_kr, w_out):
  return pl.pallas_call(
      _tc_body,
      grid=(NPT,),
      in_specs=[
          pl.BlockSpec((B, DIM), lambda i: (0, 0)),
          pl.BlockSpec((PTILE, DIM), lambda i: (i, 0)),
          pl.BlockSpec((DIM, DIM), lambda i: (0, 0)),
          pl.BlockSpec((DIM, DIM), lambda i: (0, 0)),
          pl.BlockSpec((DIM, DIM), lambda i: (0, 0)),
      ],
      out_specs=[
          pl.BlockSpec((B, PTILE), lambda i: (0, i)),
          pl.BlockSpec((PTILE, DIM), lambda i: (i, 0)),
          pl.BlockSpec((1, B, PTILE // GROUP), lambda i: (i, 0, 0)),
      ],
      out_shape=[
          jax.ShapeDtypeStruct((B, PPAD), jnp.float32),
          jax.ShapeDtypeStruct((PPAD, DIM), jnp.float32),
          jax.ShapeDtypeStruct((NPT, B, PTILE // GROUP), jnp.float32),
      ],
      scratch_shapes=[pltpu.VMEM((B, DIM), jnp.float32)],
      compiler_params=pltpu.CompilerParams(
          dimension_semantics=("arbitrary",)),
  )(h, protos_pad, w_qr, w_kr, w_out)


def _sc_topk_gather(scores, pout, gmax):
  mesh = plsc.VectorSubcoreMesh(
      core_axis_name="c", subcore_axis_name="s",
      num_cores=NC, num_subcores=NS)

  @functools.partial(
      pl.kernel,
      mesh=mesh,
      out_type=jax.ShapeDtypeStruct((B, DIM), jnp.float32),
      compiler_params=pltpu.CompilerParams(needs_layout_passes=False),
      scratch_types=[
          pltpu.VMEM((2, PPAD), jnp.float32),
          pltpu.VMEM((2, NG), jnp.float32),
          pltpu.VMEM((PPAD + LANES,), jnp.float32),
          pltpu.VMEM((PPAD + LANES,), jnp.int32),
          pltpu.VMEM((2, TOP_K), jnp.int32),
          pltpu.VMEM((2, TOP_K, DIM), jnp.float32),
          pltpu.VMEM((ROWS_PER_W, DIM), jnp.float32),
          pltpu.SemaphoreType.DMA,
          pltpu.SemaphoreType.DMA,
          pltpu.SemaphoreType.DMA,
          pltpu.SemaphoreType.DMA,
      ],
  )
  def sc_kernel(scores_hbm, pout_hbm, gmax_hbm, out_hbm,
                row_buf, gm_buf, cand_val, cand_idx, idx_v, rows_v, out_buf,
                sem0, sem1, sem_g0, sem_g1):
    wid = lax.axis_index("s") * NC + lax.axis_index("c")
    base = wid * ROWS_PER_W
    sems = (sem0, sem1)

    def fetch(r, slot):
      pltpu.async_copy(scores_hbm.at[r], row_buf.at[slot], sems[slot])
      pltpu.async_copy(gmax_hbm.at[r], gm_buf.at[slot], sems[slot])

    fetch(base, 0)
    fetch(base + 1, 1)

    def topk_row(slot):
      t = lax.sort(gm_buf[slot, pl.ds(0, LANES)])
      for c in range(1, NG // LANES):
        g = gm_buf[slot, pl.ds(c * LANES, LANES)]
        gd, _ = plsc.sort_key_val(g, g, descending=True)
        t = lax.sort(jnp.maximum(t, gd))
      thresh0 = t[0]

      lanes = lax.iota(jnp.int32, LANES)

      cnt = jnp.int32(0)
      for c in range(NG // LANES):
        gvec = gm_buf[slot, pl.ds(c * LANES, LANES)]
        mask0 = gvec >= thresh0

        def body_fn(carry, c=c):
          mask, cnt = carry
          gl = plsc.all_reduce_ffs(mask)[0]
          col0 = (c * LANES + gl) * GROUP
          for cc in range(GROUP // LANES):
            colc = col0 + cc * LANES
            v = row_buf[slot, pl.ds(colc, LANES)]
            m = v >= thresh0
            plsc.store_compressed(cand_val.at[pl.ds(cnt, LANES)], v, mask=m)
            plsc.store_compressed(cand_idx.at[pl.ds(cnt, LANES)],
                                  colc + lanes, mask=m)
            cnt = cnt + plsc.all_reduce_population_count(m)[0]
          return (mask & (lanes != gl), cnt)

        _, cnt = lax.while_loop(
            lambda carry: jnp.any(carry[0]), body_fn, (mask0, cnt))

      cand_val[pl.ds(cnt, LANES)] = jnp.full((LANES,), NEG, jnp.float32)

      def mbody(mi, carry):
        a_val, a_idx = carry
        v = cand_val[pl.ds(mi * LANES, LANES)]
        i = cand_idx[pl.ds(mi * LANES, LANES)]
        vd, idxd = plsc.sort_key_val(v, i, descending=True)
        take = vd > a_val
        nv = jnp.where(take, vd, a_val)
        ni = jnp.where(take, idxd, a_idx)
        sv, si = plsc.sort_key_val(nv, ni, descending=False)
        return (sv, si)

      a0 = jnp.full((LANES,), NEG, jnp.float32)
      i0 = jnp.zeros((LANES,), jnp.int32)
      nch = (cnt + LANES - 1) // LANES
      return lax.fori_loop(0, nch, mbody, (a0, i0))

    gsems = (sem_g0, sem_g1)

    def blend(j, w, slot):
      acc = [jnp.zeros((LANES,), jnp.float32) for _ in range(DIM // LANES)]
      for k in range(TOP_K):
        wk = w[k]
        for d in range(DIM // LANES):
          acc[d] = acc[d] + wk * rows_v[slot, k, pl.ds(d * LANES, LANES)]
      for d in range(DIM // LANES):
        out_buf[j, pl.ds(d * LANES, LANES)] = acc[d]

    def do_row(j, slot, w_prev):
      pltpu.make_async_copy(
          scores_hbm.at[base], row_buf.at[slot], sems[slot]).wait()
      pltpu.make_async_copy(
          gmax_hbm.at[base], gm_buf.at[slot], sems[slot]).wait()

      a_val, a_idx = topk_row(slot)

      @pl.when(j + 2 < ROWS_PER_W)
      def _():
        fetch(base + j + 2, slot)

      idx_v[slot, ...] = a_idx
      pltpu.async_copy(pout_hbm.at[idx_v.at[slot]], rows_v.at[slot],
                       gsems[slot])

      e = jnp.exp(a_val - jnp.max(a_val))
      w = e / jnp.sum(e)

      @pl.when(j >= 1)
      def _():
        pltpu.make_async_copy(pout_hbm.at[idx_v.at[1 - slot]],
                              rows_v.at[1 - slot], gsems[1 - slot]).wait()
        blend(j - 1, w_prev, 1 - slot)

      return w

    def pair(t, w_prev):
      w_prev = do_row(2 * t, 0, w_prev)
      w_prev = do_row(2 * t + 1, 1, w_prev)
      return w_prev

    w_last = lax.fori_loop(0, ROWS_PER_W // 2, pair,
                           jnp.zeros((LANES,), jnp.float32))
    pltpu.make_async_copy(pout_hbm.at[idx_v.at[1]], rows_v.at[1],
                          gsems[1]).wait()
    blend(ROWS_PER_W - 1, w_last, 1)

    pltpu.sync_copy(out_buf, out_hbm.at[pl.ds(base, ROWS_PER_W)])

  return sc_kernel(scores, pout, gmax)


def kernel(h, prototypes, W_QR, W_KR, W_out):
  protos_pad = jnp.pad(prototypes, ((0, PPAD - P), (0, 0)))
  scores, pout, gmax3 = _tc_scores(h, protos_pad, W_QR, W_KR, W_out)
  gmax = gmax3.transpose(1, 0, 2).reshape(B, NG)
  return _sc_topk_gather(scores, pout, gmax)


# speedup vs baseline: 5.3408x; 1.0061x over previous
import functools

import jax
import jax.numpy as jnp
from jax import lax
from jax.experimental import pallas as pl
from jax.experimental.pallas import tpu as pltpu
from jax.experimental.pallas import tpu_sc as plsc

DIM = 256
NUM_HEADS = 4
HEAD_DIM = DIM // NUM_HEADS
TOP_K = 16
TEMPERATURE = 0.1
B = 1024
P = 10000
PPAD = 10240

NC = 2
NS = 16
LANES = 16
NW = NC * NS

NCHUNKS = 2
BC = B // NCHUNKS
ROWS_PER_W = BC // NW

PTILE = 2048
NPT = PPAD // PTILE
GROUP = 128
NG = PPAD // GROUP

NEG = -3e38
SCALE = 1.0 / (NUM_HEADS * TEMPERATURE)


def _head_selector():
  d = lax.broadcasted_iota(jnp.int32, (DIM, NUM_HEADS), 0)
  h = lax.broadcasted_iota(jnp.int32, (DIM, NUM_HEADS), 1)
  return (d // HEAD_DIM == h).astype(jnp.float32)


def _head_normalize(x, sel):
  ss = lax.dot_general(x * x, sel, (((1,), (0,)), ((), ())),
                       preferred_element_type=jnp.float32,
                       precision=lax.Precision.HIGHEST)
  inv = 1.0 / jnp.maximum(jnp.sqrt(ss), 1e-12)
  inv_full = lax.dot_general(inv, sel, (((1,), (1,)), ((), ())),
                             preferred_element_type=jnp.float32,
                             precision=lax.Precision.HIGHEST)
  return x * inv_full


def _tc_body(h_ref, protos_ref, wqr_ref, wkr_ref,
             scores_ref, gmax_ref, qn_scr):
  pid = pl.program_id(0)
  sel = _head_selector()

  @pl.when(pid == 0)
  def _():
    qr = lax.dot_general(h_ref[...], wqr_ref[...], (((1,), (1,)), ((), ())),
                         preferred_element_type=jnp.float32)
    qn_scr[...] = _head_normalize(qr, sel)

  kr = lax.dot_general(protos_ref[...], wkr_ref[...], (((1,), (1,)), ((), ())),
                       preferred_element_type=jnp.float32)
  kn = _head_normalize(kr, sel)
  s = lax.dot_general(qn_scr[...], kn, (((1,), (1,)), ((), ())),
                      preferred_element_type=jnp.float32)
  col = pid * PTILE + lax.broadcasted_iota(jnp.int32, (BC, PTILE), 1)
  s = jnp.where(col < P, s * SCALE, NEG)
  scores_ref[...] = s
  gmax_ref[0] = jnp.concatenate(
      [jnp.max(s[:, g * GROUP:(g + 1) * GROUP], axis=1, keepdims=True)
       for g in range(PTILE // GROUP)], axis=1)


def _tc_scores(h_chunk, protos, w_qr, w_kr):
  return pl.pallas_call(
      _tc_body,
      grid=(NPT,),
      in_specs=[
          pl.BlockSpec((BC, DIM), lambda i: (0, 0)),
          pl.BlockSpec((PTILE, DIM), lambda i: (i, 0)),
          pl.BlockSpec((DIM, DIM), lambda i: (0, 0)),
          pl.BlockSpec((DIM, DIM), lambda i: (0, 0)),
      ],
      out_specs=[
          pl.BlockSpec((BC, PTILE), lambda i: (0, i)),
          pl.BlockSpec((1, BC, PTILE // GROUP), lambda i: (i, 0, 0)),
      ],
      out_shape=[
          jax.ShapeDtypeStruct((BC, PPAD), jnp.float32),
          jax.ShapeDtypeStruct((NPT, BC, PTILE // GROUP), jnp.float32),
      ],
      scratch_shapes=[pltpu.VMEM((BC, DIM), jnp.float32)],
      compiler_params=pltpu.CompilerParams(
          dimension_semantics=("arbitrary",)),
  )(h_chunk, protos, w_qr, w_kr)


def _tc_out_body(x_ref, w_ref, o_ref):
  o_ref[...] = lax.dot_general(x_ref[...], w_ref[...], (((1,), (1,)), ((), ())),
                               preferred_element_type=jnp.float32)


def _tc_out(x, w_out):
  return pl.pallas_call(
      _tc_out_body,
      out_shape=jax.ShapeDtypeStruct((B, DIM), jnp.float32),
  )(x, w_out)


def _sc_topk_gather(scores, protos, gmax):
  mesh = plsc.VectorSubcoreMesh(
      core_axis_name="c", subcore_axis_name="s",
      num_cores=NC, num_subcores=NS)

  @functools.partial(
      pl.kernel,
      mesh=mesh,
      out_type=jax.ShapeDtypeStruct((BC, DIM), jnp.float32),
      compiler_params=pltpu.CompilerParams(needs_layout_passes=False),
      scratch_types=[
          pltpu.VMEM((2, PPAD), jnp.float32),
          pltpu.VMEM((2, NG), jnp.float32),
          pltpu.VMEM((PPAD + LANES,), jnp.float32),
          pltpu.VMEM((PPAD + LANES,), jnp.int32),
          pltpu.VMEM((2, TOP_K), jnp.int32),
          pltpu.VMEM((2, TOP_K, DIM), jnp.float32),
          pltpu.VMEM((ROWS_PER_W, DIM), jnp.float32),
          pltpu.SemaphoreType.DMA,
          pltpu.SemaphoreType.DMA,
          pltpu.SemaphoreType.DMA,
          pltpu.SemaphoreType.DMA,
      ],
  )
  def sc_kernel(scores_hbm, protos_hbm, gmax_hbm, out_hbm,
                row_buf, gm_buf, cand_val, cand_idx, idx_v, rows_v, out_buf,
                sem0, sem1, sem_g0, sem_g1):
    wid = lax.axis_index("s") * NC + lax.axis_index("c")
    base = wid * ROWS_PER_W
    sems = (sem0, sem1)

    def fetch(r, slot):
      pltpu.async_copy(scores_hbm.at[r], row_buf.at[slot], sems[slot])
      pltpu.async_copy(gmax_hbm.at[r], gm_buf.at[slot], sems[slot])

    fetch(base, 0)
    fetch(base + 1, 1)

    def topk_row(slot):
      t = lax.sort(gm_buf[slot, pl.ds(0, LANES)])
      for c in range(1, NG // LANES):
        g = gm_buf[slot, pl.ds(c * LANES, LANES)]
        gd, _ = plsc.sort_key_val(g, g, descending=True)
        t = lax.sort(jnp.maximum(t, gd))
      thresh0 = t[0]

      lanes = lax.iota(jnp.int32, LANES)

      cnt = jnp.int32(0)
      for c in range(NG // LANES):
        gvec = gm_buf[slot, pl.ds(c * LANES, LANES)]
        mask0 = gvec >= thresh0

        def body_fn(carry, c=c):
          mask, cnt = carry
          gl = plsc.all_reduce_ffs(mask)[0]
          col0 = (c * LANES + gl) * GROUP
          for cc in range(GROUP // LANES):
            colc = col0 + cc * LANES
            v = row_buf[slot, pl.ds(colc, LANES)]
            m = v >= thresh0
            plsc.store_compressed(cand_val.at[pl.ds(cnt, LANES)], v, mask=m)
            plsc.store_compressed(cand_idx.at[pl.ds(cnt, LANES)],
                                  colc + lanes, mask=m)
            cnt = cnt + plsc.all_reduce_population_count(m)[0]
          return (mask & (lanes != gl), cnt)

        _, cnt = lax.while_loop(
            lambda carry: jnp.any(carry[0]), body_fn, (mask0, cnt))

      cand_val[pl.ds(cnt, LANES)] = jnp.full((LANES,), NEG, jnp.float32)

      def mbody(mi, carry):
        a_val, a_idx = carry
        v = cand_val[pl.ds(mi * LANES, LANES)]
        i = cand_idx[pl.ds(mi * LANES, LANES)]
        vd, idxd = plsc.sort_key_val(v, i, descending=True)
        take = vd > a_val
        nv = jnp.where(take, vd, a_val)
        ni = jnp.where(take, idxd, a_idx)
        sv, si = plsc.sort_key_val(nv, ni, descending=False)
        return (sv, si)

      a0 = jnp.full((LANES,), NEG, jnp.float32)
      i0 = jnp.zeros((LANES,), jnp.int32)
      nch = (cnt + LANES - 1) // LANES
      return lax.fori_loop(0, nch, mbody, (a0, i0))

    gsems = (sem_g0, sem_g1)

    def blend(j, w, slot):
      acc = [jnp.zeros((LANES,), jnp.float32) for _ in range(DIM // LANES)]
      for k in range(TOP_K):
        wk = w[k]
        for d in range(DIM // LANES):
          acc[d] = acc[d] + wk * rows_v[slot, k, pl.ds(d * LANES, LANES)]
      for d in range(DIM // LANES):
        out_buf[j, pl.ds(d * LANES, LANES)] = acc[d]

    def do_row(j, slot, w_prev):
      pltpu.make_async_copy(
          scores_hbm.at[base], row_buf.at[slot], sems[slot]).wait()
      pltpu.make_async_copy(
          gmax_hbm.at[base], gm_buf.at[slot], sems[slot]).wait()

      a_val, a_idx = topk_row(slot)

      @pl.when(j + 2 < ROWS_PER_W)
      def _():
        fetch(base + j + 2, slot)

      idx_v[slot, ...] = a_idx
      pltpu.async_copy(protos_hbm.at[idx_v.at[slot]], rows_v.at[slot],
                       gsems[slot])

      e = jnp.exp(a_val - jnp.max(a_val))
      w = e / jnp.sum(e)

      @pl.when(j >= 1)
      def _():
        pltpu.make_async_copy(protos_hbm.at[idx_v.at[1 - slot]],
                              rows_v.at[1 - slot], gsems[1 - slot]).wait()
        blend(j - 1, w_prev, 1 - slot)

      return w

    def pair(t, w_prev):
      w_prev = do_row(2 * t, 0, w_prev)
      w_prev = do_row(2 * t + 1, 1, w_prev)
      return w_prev

    w_last = lax.fori_loop(0, ROWS_PER_W // 2, pair,
                           jnp.zeros((LANES,), jnp.float32))
    pltpu.make_async_copy(protos_hbm.at[idx_v.at[1]], rows_v.at[1],
                          gsems[1]).wait()
    blend(ROWS_PER_W - 1, w_last, 1)

    pltpu.sync_copy(out_buf, out_hbm.at[pl.ds(base, ROWS_PER_W)])

  return sc_kernel(scores, protos, gmax)


def kernel(h, prototypes, W_QR, W_KR, W_out):
  retrieved = []
  for ci in range(NCHUNKS):
    h_chunk = lax.slice(h, (ci * BC, 0), ((ci + 1) * BC, DIM))
    scores, gmax3 = _tc_scores(h_chunk, prototypes, W_QR, W_KR)
    gmax = gmax3.transpose(1, 0, 2).reshape(BC, NG)
    retrieved.append(_sc_topk_gather(scores, prototypes, gmax))
  return _tc_out(jnp.concatenate(retrieved, axis=0), W_out)
